# baseline (speedup 1.0000x reference)
import functools

import jax
import jax.numpy as jnp
from jax import lax
from jax.experimental import pallas as pl
from jax.experimental.pallas import tpu as pltpu
from jax.experimental.pallas import tpu_sc as plsc

N_AGENTS = 10000
DEG = 16
E = N_AGENTS * DEG
NBR_IN = 256
AGENT_IN = 256
HIDDEN = 256
NBR_OUT = 8
AGENT_OUT = 8
MAX_NBRS = 32
_PK = HIDDEN // 2


_BA = 400


def _pre_body(xn_ref, xa_ref, w1a_ref, w1b_ref, b1_ref, wa_ref, ba_ref,
              sp_ref, ap_ref, own_ref):
    xn = xn_ref[...]
    xa = xa_ref[...]
    dot = functools.partial(lax.dot_general,
                            dimension_numbers=(((1,), (0,)), ((), ())),
                            preferred_element_type=jnp.float32)
    a = dot(xn, w1a_ref[...])
    u_hi = lax.bitcast_convert_type(
        a[:, _PK:].astype(jnp.bfloat16).astype(jnp.float32), jnp.uint32)
    u_lo = lax.bitcast_convert_type(
        a[:, :_PK].astype(jnp.bfloat16).astype(jnp.float32), jnp.uint32)
    sp_ref[...] = u_hi | (u_lo >> 16)
    ap_ref[...] = dot(xa, w1b_ref[...]) + b1_ref[...]
    own_ref[...] = dot(xa, wa_ref[...]) + ba_ref[...]


def _precompute(x_nbr, x_agent, w1a, w1b, b1, wa, ba):
    grid = N_AGENTS // _BA
    return pl.pallas_call(
        _pre_body,
        grid=(grid,),
        in_specs=[
            pl.BlockSpec((_BA, NBR_IN), lambda i: (i, 0)),
            pl.BlockSpec((_BA, AGENT_IN), lambda i: (i, 0)),
            pl.BlockSpec((NBR_IN, HIDDEN), lambda i: (0, 0)),
            pl.BlockSpec((AGENT_IN, HIDDEN), lambda i: (0, 0)),
            pl.BlockSpec((1, HIDDEN), lambda i: (0, 0)),
            pl.BlockSpec((AGENT_IN, AGENT_OUT), lambda i: (0, 0)),
            pl.BlockSpec((1, AGENT_OUT), lambda i: (0, 0)),
        ],
        out_specs=[
            pl.BlockSpec((_BA, _PK), lambda i: (i, 0)),
            pl.BlockSpec((_BA, HIDDEN), lambda i: (i, 0)),
            pl.BlockSpec((_BA, AGENT_OUT), lambda i: (i, 0)),
        ],
        out_shape=[
            jax.ShapeDtypeStruct((N_AGENTS, _PK), jnp.uint32),
            jax.ShapeDtypeStruct((N_AGENTS, HIDDEN), jnp.float32),
            jax.ShapeDtypeStruct((N_AGENTS, AGENT_OUT), jnp.float32),
        ],
    )(x_nbr, x_agent, w1a, w1b, b1, wa, ba)


_NW = 32
_CH = 128


def _gather_body(sp_hbm, src_hbm, out_hbm,
                 idx_all, rows0, rows1, rows2, rowst_v,
                 gsem0, gsem1, gsem2, wsem0, wsem1, wsem2, tsem,
                 *, off, epw, nfull, tail):
    wid = lax.axis_index("s") * 2 + lax.axis_index("c")
    base_w = wid * epw
    bufs = (rows0, rows1, rows2)
    gsems = (gsem0, gsem1, gsem2)
    wsems = (wsem0, wsem1, wsem2)

    pltpu.sync_copy(src_hbm.at[pl.ds(off + base_w, epw)], idx_all)

    gops = [None] * nfull
    wops = [None] * nfull
    for c in range(nfull + 2):
        if c < nfull:
            b = c % 3
            if c >= 3:
                wops[c - 3].wait()
            idx_v = idx_all.at[pl.ds(c * _CH, _CH)]
            gops[c] = pltpu.async_copy(sp_hbm.at[idx_v], bufs[b], gsems[b])
        if c >= 2:
            cc = c - 2
            b2 = cc % 3
            gops[cc].wait()
            wops[cc] = pltpu.async_copy(
                bufs[b2], out_hbm.at[pl.ds(base_w + cc * _CH, _CH)],
                wsems[b2])

    if tail:
        base = base_w + nfull * _CH
        idx_t = idx_all.at[pl.ds(nfull * _CH, tail)]
        pltpu.async_copy(sp_hbm.at[idx_t], rowst_v, tsem).wait()
        pltpu.sync_copy(rowst_v, out_hbm.at[pl.ds(base, tail)])

    wops[nfull - 3].wait()
    wops[nfull - 2].wait()
    wops[nfull - 1].wait()


def _gather(sp, edge_src, off, ec):
    epw = ec // _NW
    nfull = epw // _CH
    tail = epw - nfull * _CH
    mesh = plsc.VectorSubcoreMesh(core_axis_name="c", subcore_axis_name="s")
    k = functools.partial(
        pl.kernel,
        out_type=jax.ShapeDtypeStruct((ec, _PK), jnp.uint32),
        mesh=mesh,
        scratch_types=[
            pltpu.VMEM((epw,), jnp.int32),
            pltpu.VMEM((_CH, _PK), jnp.uint32),
            pltpu.VMEM((_CH, _PK), jnp.uint32),
            pltpu.VMEM((_CH, _PK), jnp.uint32),
            pltpu.VMEM((max(tail, 1), _PK), jnp.uint32),
            pltpu.SemaphoreType.DMA,
            pltpu.SemaphoreType.DMA,
            pltpu.SemaphoreType.DMA,
            pltpu.SemaphoreType.DMA,
            pltpu.SemaphoreType.DMA,
            pltpu.SemaphoreType.DMA,
            pltpu.SemaphoreType.DMA,
        ],
    )(functools.partial(_gather_body, off=off, epw=epw,
                        nfull=nfull, tail=tail))
    return k(sp, edge_src)


_BAGENTS = 200
_BE = _BAGENTS * DEG


_OUT_W = MAX_NBRS * NBR_OUT + AGENT_OUT


_DN = DEG * NBR_OUT


def _mlp_body(g_ref, ap_ref, own_ref, w2l_ref, w2h_ref, b2_ref, out_ref):
    g32 = g_ref[...]
    ap = ap_ref[...]
    h_lo = lax.bitcast_convert_type(g32 << jnp.uint32(16), jnp.float32)
    h_hi = lax.bitcast_convert_type(g32 & jnp.uint32(0xFFFF0000), jnp.float32)
    hl = jnp.maximum(h_lo.reshape(_BAGENTS, DEG, _PK) + ap[:, None, :_PK],
                     0.0).reshape(_BE, _PK).astype(jnp.bfloat16)
    hh = jnp.maximum(h_hi.reshape(_BAGENTS, DEG, _PK) + ap[:, None, _PK:],
                     0.0).reshape(_BE, _PK).astype(jnp.bfloat16)
    dot = functools.partial(lax.dot_general,
                            dimension_numbers=(((1,), (0,)), ((), ())),
                            preferred_element_type=jnp.float32)
    d = dot(hl, w2l_ref[...]) + dot(hh, w2h_ref[...])
    d3 = (d + b2_ref[...]).reshape(_BAGENTS, DEG, _DN)
    lane = lax.broadcasted_iota(jnp.int32, (DEG, _DN), 1) // NBR_OUT
    slot = lax.broadcasted_iota(jnp.int32, (DEG, _DN), 0)
    mask = (lane == slot).astype(jnp.float32)
    scores = jnp.sum(d3 * mask[None], axis=1)
    out_ref[...] = jnp.concatenate(
        [scores,
         jnp.zeros((_BAGENTS, (MAX_NBRS - DEG) * NBR_OUT), jnp.float32),
         own_ref[...]], axis=1)


def _mlp_tail(g, ap, own, w2l, w2h, b2, agent_off, ec):
    grid = ec // _BE
    aoff = agent_off // _BAGENTS
    return pl.pallas_call(
        _mlp_body,
        grid=(grid,),
        in_specs=[
            pl.BlockSpec((_BE, _PK), lambda i: (i, 0)),
            pl.BlockSpec((_BAGENTS, HIDDEN), lambda i: (i + aoff, 0)),
            pl.BlockSpec((_BAGENTS, AGENT_OUT), lambda i: (i + aoff, 0)),
            pl.BlockSpec((_PK, _DN), lambda i: (0, 0)),
            pl.BlockSpec((_PK, _DN), lambda i: (0, 0)),
            pl.BlockSpec((1, _DN), lambda i: (0, 0)),
        ],
        out_specs=pl.BlockSpec((_BAGENTS, _OUT_W), lambda i: (i, 0)),
        out_shape=jax.ShapeDtypeStruct((ec // DEG, _OUT_W), jnp.float32),
    )(g, ap, own, w2l, w2h, b2)


def kernel(x_nbr, x_agent, edge_src, edge_dst, W1, b1, W2, b2, Wa, ba):
    w1a = W1[:NBR_IN]
    w1b = W1[NBR_IN:]
    sp, ap, own = _precompute(x_nbr, x_agent, w1a, w1b,
                              b1.reshape(1, HIDDEN), Wa,
                              ba.reshape(1, AGENT_OUT))
    src32 = edge_src.astype(jnp.int32)
    w2l = jnp.tile(W2[:_PK], (1, DEG)).astype(jnp.bfloat16)
    w2h = jnp.tile(W2[_PK:], (1, DEG)).astype(jnp.bfloat16)
    b2rep = jnp.tile(b2.reshape(1, NBR_OUT), (1, DEG))
    g = _gather(sp, src32, 0, E)
    return _mlp_tail(g, ap, own, w2l, w2h, b2rep, 0, E)


# speedup vs baseline: 1.0099x; 1.0099x over previous
import functools

import jax
import jax.numpy as jnp
from jax import lax
from jax.experimental import pallas as pl
from jax.experimental.pallas import tpu as pltpu
from jax.experimental.pallas import tpu_sc as plsc

N_AGENTS = 10000
DEG = 16
E = N_AGENTS * DEG
NBR_IN = 256
AGENT_IN = 256
HIDDEN = 256
NBR_OUT = 8
AGENT_OUT = 8
MAX_NBRS = 32
_PK = HIDDEN // 2


_BA = 400


def _pre_body(xn_ref, xa_ref, w1a_ref, w1b_ref, b1_ref, wa_ref, ba_ref,
              sp_ref, ap_ref, own_ref):
    xn = xn_ref[...]
    xa = xa_ref[...]
    dot = functools.partial(lax.dot_general,
                            dimension_numbers=(((1,), (0,)), ((), ())),
                            preferred_element_type=jnp.float32)
    a = dot(xn, w1a_ref[...])
    u_hi = lax.bitcast_convert_type(
        a[:, _PK:].astype(jnp.bfloat16).astype(jnp.float32), jnp.uint32)
    u_lo = lax.bitcast_convert_type(
        a[:, :_PK].astype(jnp.bfloat16).astype(jnp.float32), jnp.uint32)
    sp_ref[...] = u_hi | (u_lo >> 16)
    ap_ref[...] = dot(xa, w1b_ref[...]) + b1_ref[...]
    own_ref[...] = dot(xa, wa_ref[...]) + ba_ref[...]


def _precompute(x_nbr, x_agent, w1a, w1b, b1, wa, ba):
    grid = N_AGENTS // _BA
    return pl.pallas_call(
        _pre_body,
        grid=(grid,),
        in_specs=[
            pl.BlockSpec((_BA, NBR_IN), lambda i: (i, 0)),
            pl.BlockSpec((_BA, AGENT_IN), lambda i: (i, 0)),
            pl.BlockSpec((NBR_IN, HIDDEN), lambda i: (0, 0)),
            pl.BlockSpec((AGENT_IN, HIDDEN), lambda i: (0, 0)),
            pl.BlockSpec((1, HIDDEN), lambda i: (0, 0)),
            pl.BlockSpec((AGENT_IN, AGENT_OUT), lambda i: (0, 0)),
            pl.BlockSpec((1, AGENT_OUT), lambda i: (0, 0)),
        ],
        out_specs=[
            pl.BlockSpec((_BA, _PK), lambda i: (i, 0)),
            pl.BlockSpec((_BA, HIDDEN), lambda i: (i, 0)),
            pl.BlockSpec((_BA, AGENT_OUT), lambda i: (i, 0)),
        ],
        out_shape=[
            jax.ShapeDtypeStruct((N_AGENTS, _PK), jnp.uint32),
            jax.ShapeDtypeStruct((N_AGENTS, HIDDEN), jnp.float32),
            jax.ShapeDtypeStruct((N_AGENTS, AGENT_OUT), jnp.float32),
        ],
    )(x_nbr, x_agent, w1a, w1b, b1, wa, ba)


_NW = 32
_CH = 128


def _gather_body(sp_hbm, src_hbm, out_hbm,
                 idx_all, rows0, rows1, rows2, rowst_v,
                 gsem0, gsem1, gsem2, wsem0, wsem1, wsem2, tsem,
                 *, off, epw, nfull, tail):
    wid = lax.axis_index("s") * 2 + lax.axis_index("c")
    base_w = wid * epw
    bufs = (rows0, rows1, rows2)
    gsems = (gsem0, gsem1, gsem2)
    wsems = (wsem0, wsem1, wsem2)

    pltpu.sync_copy(src_hbm.at[pl.ds(off + base_w, epw)], idx_all)

    gops = [None] * nfull
    wops = [None] * nfull
    for c in range(nfull + 2):
        if c < nfull:
            b = c % 3
            if c >= 3:
                wops[c - 3].wait()
            idx_v = idx_all.at[pl.ds(c * _CH, _CH)]
            gops[c] = pltpu.async_copy(sp_hbm.at[idx_v], bufs[b], gsems[b])
        if c >= 2:
            cc = c - 2
            b2 = cc % 3
            gops[cc].wait()
            wops[cc] = pltpu.async_copy(
                bufs[b2], out_hbm.at[pl.ds(base_w + cc * _CH, _CH)],
                wsems[b2])

    if tail:
        base = base_w + nfull * _CH
        idx_t = idx_all.at[pl.ds(nfull * _CH, tail)]
        pltpu.async_copy(sp_hbm.at[idx_t], rowst_v, tsem).wait()
        pltpu.sync_copy(rowst_v, out_hbm.at[pl.ds(base, tail)])

    wops[nfull - 3].wait()
    wops[nfull - 2].wait()
    wops[nfull - 1].wait()


def _gather(sp, edge_src, off, ec):
    epw = ec // _NW
    nfull = epw // _CH
    tail = epw - nfull * _CH
    mesh = plsc.VectorSubcoreMesh(core_axis_name="c", subcore_axis_name="s")
    k = functools.partial(
        pl.kernel,
        out_type=jax.ShapeDtypeStruct((ec, _PK), jnp.uint32),
        mesh=mesh,
        scratch_types=[
            pltpu.VMEM((epw,), jnp.int32),
            pltpu.VMEM((_CH, _PK), jnp.uint32),
            pltpu.VMEM((_CH, _PK), jnp.uint32),
            pltpu.VMEM((_CH, _PK), jnp.uint32),
            pltpu.VMEM((max(tail, 1), _PK), jnp.uint32),
            pltpu.SemaphoreType.DMA,
            pltpu.SemaphoreType.DMA,
            pltpu.SemaphoreType.DMA,
            pltpu.SemaphoreType.DMA,
            pltpu.SemaphoreType.DMA,
            pltpu.SemaphoreType.DMA,
            pltpu.SemaphoreType.DMA,
        ],
    )(functools.partial(_gather_body, off=off, epw=epw,
                        nfull=nfull, tail=tail))
    return k(sp, edge_src)


_BAGENTS = 200
_BE = _BAGENTS * DEG


_OUT_W = MAX_NBRS * NBR_OUT + AGENT_OUT


_DN = DEG * NBR_OUT


def _mlp_body(g_ref, ap_ref, own_ref, w2l_ref, w2h_ref, b2_ref, out_ref):
    g32 = g_ref[...]
    ap = ap_ref[...]
    h_lo = lax.bitcast_convert_type(g32 << jnp.uint32(16), jnp.float32)
    h_hi = lax.bitcast_convert_type(g32 & jnp.uint32(0xFFFF0000), jnp.float32)
    hl = jnp.maximum(h_lo.reshape(_BAGENTS, DEG, _PK) + ap[:, None, :_PK],
                     0.0).reshape(_BE, _PK)
    hh = jnp.maximum(h_hi.reshape(_BAGENTS, DEG, _PK) + ap[:, None, _PK:],
                     0.0).reshape(_BE, _PK)
    dot = functools.partial(lax.dot_general,
                            dimension_numbers=(((1,), (0,)), ((), ())),
                            preferred_element_type=jnp.float32)
    d = dot(hl, w2l_ref[...]) + dot(hh, w2h_ref[...])
    d3 = (d + b2_ref[...]).reshape(_BAGENTS, DEG, _DN)
    lane = lax.broadcasted_iota(jnp.int32, (DEG, _DN), 1) // NBR_OUT
    slot = lax.broadcasted_iota(jnp.int32, (DEG, _DN), 0)
    mask = (lane == slot).astype(jnp.float32)
    scores = jnp.sum(d3 * mask[None], axis=1)
    out_ref[...] = jnp.concatenate(
        [scores,
         jnp.zeros((_BAGENTS, (MAX_NBRS - DEG) * NBR_OUT), jnp.float32),
         own_ref[...]], axis=1)


def _mlp_tail(g, ap, own, w2l, w2h, b2, agent_off, ec):
    grid = ec // _BE
    aoff = agent_off // _BAGENTS
    return pl.pallas_call(
        _mlp_body,
        grid=(grid,),
        in_specs=[
            pl.BlockSpec((_BE, _PK), lambda i: (i, 0)),
            pl.BlockSpec((_BAGENTS, HIDDEN), lambda i: (i + aoff, 0)),
            pl.BlockSpec((_BAGENTS, AGENT_OUT), lambda i: (i + aoff, 0)),
            pl.BlockSpec((_PK, _DN), lambda i: (0, 0)),
            pl.BlockSpec((_PK, _DN), lambda i: (0, 0)),
            pl.BlockSpec((1, _DN), lambda i: (0, 0)),
        ],
        out_specs=pl.BlockSpec((_BAGENTS, _OUT_W), lambda i: (i, 0)),
        out_shape=jax.ShapeDtypeStruct((ec // DEG, _OUT_W), jnp.float32),
    )(g, ap, own, w2l, w2h, b2)


def kernel(x_nbr, x_agent, edge_src, edge_dst, W1, b1, W2, b2, Wa, ba):
    w1a = W1[:NBR_IN]
    w1b = W1[NBR_IN:]
    sp, ap, own = _precompute(x_nbr, x_agent, w1a, w1b,
                              b1.reshape(1, HIDDEN), Wa,
                              ba.reshape(1, AGENT_OUT))
    src32 = edge_src.astype(jnp.int32)
    w2l = jnp.tile(W2[:_PK], (1, DEG))
    w2h = jnp.tile(W2[_PK:], (1, DEG))
    b2rep = jnp.tile(b2.reshape(1, NBR_OUT), (1, DEG))
    g = _gather(sp, src32, 0, E)
    return _mlp_tail(g, ap, own, w2l, w2h, b2rep, 0, E)
